# baseline (speedup 1.0000x reference)
import functools

import jax
import jax.numpy as jnp
from jax import lax
from jax.experimental import pallas as pl
from jax.experimental.pallas import tpu as pltpu
from jax.experimental.pallas import tpu_sc as plsc

EMBEDDING_LENGTH = 128
VOCAB = 256

NUM_CORES = 2
NUM_SUBCORES = 16
NW = NUM_CORES * NUM_SUBCORES

CHUNK = 128
NGRP = 3
NBUF = 2 * NGRP


def _make_kernel(n_rows: int):
    assert n_rows % (NW * CHUNK) == 0
    chunks_per_w = n_rows // (NW * CHUNK)
    assert chunks_per_w % 2 == 0 and chunks_per_w >= 4 * NGRP
    mesh = plsc.VectorSubcoreMesh(
        core_axis_name="c", subcore_axis_name="s",
        num_cores=NUM_CORES, num_subcores=NUM_SUBCORES)

    @functools.partial(
        pl.kernel,
        out_type=jax.ShapeDtypeStruct((n_rows, EMBEDDING_LENGTH), jnp.float32),
        mesh=mesh,
        scratch_types=[
            pltpu.VMEM((chunks_per_w, CHUNK), jnp.int32),
        ] + [pltpu.VMEM((CHUNK, EMBEDDING_LENGTH), jnp.float32)] * NBUF
          + [pltpu.SemaphoreType.DMA] * (2 * NBUF),
    )
    def gather_kernel(table_hbm, idx_hbm, out_hbm, idx_v, *rest):
        bufs = rest[:NBUF]
        gsems = rest[NBUF:2 * NBUF]
        wsems = rest[2 * NBUF:]
        sid = lax.axis_index("s")
        wid = sid * NUM_CORES + lax.axis_index("c")
        base = wid * chunks_per_w

        pltpu.sync_copy(idx_hbm.at[pl.ds(base, chunks_per_w)], idx_v)

        def out_slice(j):
            return out_hbm.at[pl.ds((base + j) * CHUNK, CHUNK)]

        def fire_pair(g, grp):
            return [
                pltpu.async_copy(
                    table_hbm.at[idx_v.at[2 * g + i]],
                    bufs[2 * grp + i], gsems[2 * grp + i])
                for i in range(2)
            ]

        def finish_pair(g, grp, handles):
            for i in range(2):
                handles[i].wait()
                pltpu.async_copy(
                    bufs[2 * grp + i], out_slice(2 * g + i),
                    wsems[2 * grp + i])

        def drain_pair(g, grp):
            for i in range(2):
                pltpu.make_async_copy(
                    bufs[2 * grp + i], out_slice(2 * g + i),
                    wsems[2 * grp + i]).wait()

        n_pairs = chunks_per_w // 2
        n_steady = (n_pairs - NGRP) // NGRP
        n_tail = n_pairs - NGRP - n_steady * NGRP

        for grp in range(NGRP):
            finish_pair(grp, grp, fire_pair(grp, grp))

        def body(k, carry):
            p0 = NGRP + k * NGRP
            hs = []
            for grp in range(NGRP):
                drain_pair(p0 + grp - NGRP, grp)
                hs.append(fire_pair(p0 + grp, grp))
            for grp in range(NGRP):
                finish_pair(p0 + grp, grp, hs[grp])
            return carry

        lax.fori_loop(0, n_steady, body, 0)

        for t in range(n_tail):
            p = NGRP + n_steady * NGRP + t
            grp = p % NGRP
            drain_pair(p - NGRP, grp)
            finish_pair(p, grp, fire_pair(p, grp))

        for p in range(n_pairs - NGRP, n_pairs):
            drain_pair(p, p % NGRP)

    return gather_kernel


def kernel(x, table):
    orig_shape = x.shape
    n_rows = x.size
    idx = x.reshape(n_rows // CHUNK, CHUNK).astype(jnp.int32)
    out = _make_kernel(n_rows)(table, idx)
    return out.reshape(*orig_shape, EMBEDDING_LENGTH)


# speedup vs baseline: 3.8817x; 3.8817x over previous
import functools

import jax
import jax.numpy as jnp
from jax import lax
from jax.experimental import pallas as pl
from jax.experimental.pallas import tpu as pltpu
from jax.experimental.pallas import tpu_sc as plsc

EMBEDDING_LENGTH = 128
VOCAB = 256

NUM_CORES = 2
NUM_SUBCORES = 16
NW = NUM_CORES * NUM_SUBCORES

CHUNK = 64
RING = 8
AHEAD = 4


def _make_kernel(n_rows: int):
    assert n_rows % (NW * CHUNK) == 0
    chunks_per_w = n_rows // (NW * CHUNK)
    assert chunks_per_w % RING == 0 and chunks_per_w >= 2 * RING
    mesh = plsc.VectorSubcoreMesh(
        core_axis_name="c", subcore_axis_name="s",
        num_cores=NUM_CORES, num_subcores=NUM_SUBCORES)

    @functools.partial(
        pl.kernel,
        out_type=jax.ShapeDtypeStruct((n_rows, EMBEDDING_LENGTH), jnp.float32),
        mesh=mesh,
        scratch_types=[
            pltpu.VMEM((chunks_per_w, CHUNK), jnp.int32),
            pltpu.VMEM_SHARED((VOCAB, EMBEDDING_LENGTH), jnp.float32),
        ] + [pltpu.VMEM((CHUNK, EMBEDDING_LENGTH), jnp.float32)] * RING
          + [pltpu.SemaphoreType.DMA] * (2 * RING),
    )
    def gather_kernel(table_hbm, idx_hbm, out_hbm, idx_v, table_v, *rest):
        bufs = rest[:RING]
        gsems = rest[RING:2 * RING]
        wsems = rest[2 * RING:]
        sid = lax.axis_index("s")
        wid = sid * NUM_CORES + lax.axis_index("c")
        base = wid * chunks_per_w

        pltpu.sync_copy(idx_hbm.at[pl.ds(base, chunks_per_w)], idx_v)

        @pl.when(sid == 0)
        def _stage_table():
            pltpu.sync_copy(table_hbm, table_v)

        plsc.subcore_barrier()

        def out_slice(c):
            return out_hbm.at[pl.ds((base + c) * CHUNK, CHUNK)]

        def fire_gather(c, b):
            pltpu.async_copy(table_v.at[idx_v.at[c]], bufs[b], gsems[b])

        def wait_gather(c, b):
            pltpu.make_async_copy(
                table_v.at[idx_v.at[c]], bufs[b], gsems[b]).wait()

        def fire_write(c, b):
            pltpu.async_copy(bufs[b], out_slice(c), wsems[b])

        def drain_write(c, b):
            pltpu.make_async_copy(bufs[b], out_slice(c), wsems[b]).wait()

        for c in range(AHEAD):
            fire_gather(c, c % RING)

        def body(k, carry):
            c0 = RING * k
            for i in range(RING):
                c = c0 + i
                pa = c + AHEAD
                ba = (i + AHEAD) % RING
                drain_write(pa - RING, ba)
                fire_gather(pa, ba)
                wait_gather(c, i)
                fire_write(c, i)
            return carry

        for i in range(RING):
            c = i
            pa = c + AHEAD
            ba = (i + AHEAD) % RING
            if pa - RING >= 0:
                drain_write(pa - RING, ba)
            fire_gather(pa, ba)
            wait_gather(c, i)
            fire_write(c, i)

        lax.fori_loop(1, chunks_per_w // RING - 1, body, 0)

        c0 = chunks_per_w - RING
        for i in range(RING):
            c = c0 + i
            pa = c + AHEAD
            ba = (i + AHEAD) % RING
            if pa < chunks_per_w:
                drain_write(pa - RING, ba)
                fire_gather(pa, ba)
            wait_gather(c, i)
            fire_write(c, i)

        for c in range(chunks_per_w - RING, chunks_per_w):
            drain_write(c, c % RING)

    return gather_kernel


def kernel(x, table):
    orig_shape = x.shape
    n_rows = x.size
    idx = x.reshape(n_rows // CHUNK, CHUNK).astype(jnp.int32)
    out = _make_kernel(n_rows)(table, idx)
    return out.reshape(*orig_shape, EMBEDDING_LENGTH)
